# baseline (speedup 1.0000x reference)
import functools

import jax
import jax.numpy as jnp
from jax import lax
from jax.experimental import pallas as pl
from jax.experimental.pallas import tpu as pltpu
from jax.experimental.pallas import tpu_sc as plsc

_MINIMUM = 0.3
_MAXIMUM = 0.9

_NS = 16
_LANES = 16
_BAND = 16
_NBANDS = 3
_SC_ROWS = _BAND * _NBANDS * 4
_UNROLL = 4


def _sc_stage(logits, masks, B, C, N, W, H):
    Cf = C - 1
    kvecs = H // _LANES

    mesh = plsc.VectorSubcoreMesh(core_axis_name="c", subcore_axis_name="s")
    psize = N * 2 * _LANES

    @functools.partial(
        pl.kernel,
        mesh=mesh,
        out_type=(jax.ShapeDtypeStruct((_NS, psize), jnp.float32),
                  jax.ShapeDtypeStruct((_NS, psize), jnp.float32)),
        scratch_types=[
            pltpu.VMEM((2, _BAND, H), jnp.float32),
            pltpu.VMEM((2, N, _BAND, H), jnp.float32),
            pltpu.VMEM((psize,), jnp.float32),
            pltpu.SemaphoreType.DMA,
            pltpu.SemaphoreType.DMA,
            pltpu.SemaphoreType.DMA,
            pltpu.SemaphoreType.DMA,
        ],
        compiler_params=pltpu.CompilerParams(use_tc_tiling_on_sc=True),
    )
    def sc_kernel(l_hbm, m_hbm, out0_hbm, out1_hbm, lbuf, mbuf, acc,
                  sl0, sl1, sm0, sm1):
        ci = lax.axis_index("c")
        sid = lax.axis_index("s")
        wid = ci * _NS + sid
        bc = wid // 4
        q = wid % 4
        b = bc // Cf
        c = 1 + bc % Cf
        r_base = q * (_BAND * _NBANDS)
        lsems = (sl0, sl1)
        msems = (sm0, sm1)

        def issue(u, slot):
            r0 = r_base + u * _BAND
            cl = pltpu.async_copy(
                l_hbm.at[b, c, pl.ds(r0, _BAND), :], lbuf.at[slot],
                lsems[slot])
            cm = pltpu.async_copy(
                m_hbm.at[b, c, :, pl.ds(r0, _BAND), :], mbuf.at[slot],
                msems[slot])
            return cl, cm

        zero = jnp.zeros((_LANES,), jnp.float32)
        accs = (zero,) * (2 * N)
        pend = issue(0, 0)
        for u in range(_NBANDS):
            slot = u % 2
            cl, cm = pend
            if u + 1 < _NBANDS:
                pend = issue(u + 1, 1 - slot)
            cl.wait()
            cm.wait()

            for r in range(_BAND):
                def v_body(k, carry, _r=r, _slot=slot):
                    s1s = list(carry[:N])
                    s2s = list(carry[N:])
                    for uu in range(_UNROLL):
                        off = (k * _UNROLL + uu) * _LANES
                        lv = lbuf[_slot, _r, pl.ds(off, _LANES)]
                        for n in range(N):
                            m = mbuf[_slot, n, _r, pl.ds(off, _LANES)]
                            s1s[n] = s1s[n] + m
                            s2s[n] = s2s[n] + m * lv
                    return tuple(s1s) + tuple(s2s)

                accs = lax.fori_loop(0, kvecs // _UNROLL, v_body, accs)

        for n in range(N):
            acc[pl.ds(n * 2 * _LANES, _LANES)] = accs[n]
            acc[pl.ds(n * 2 * _LANES + _LANES, _LANES)] = accs[N + n]

        @pl.when(ci == 0)
        def _w0():
            pltpu.sync_copy(acc, out0_hbm.at[sid])

        @pl.when(ci == 1)
        def _w1():
            pltpu.sync_copy(acc, out1_hbm.at[sid])

    return sc_kernel(logits, masks)


def _tc_main_body(l_ref, m_ref, out_ref):
    l = l_ref[0, 0]
    m = m_ref[0, 0]
    out_ref[0, 0, :, 0] = jnp.sum(m, axis=(1, 2))
    out_ref[0, 0, :, 1] = jnp.sum(m * l[None, :, :], axis=(1, 2))


def _combine_body(p0_ref, p1_ref, t_ref, out_ref):
    def sc_sums(p):
        box = jnp.sum(p[:, :, :, 0, :], axis=(1, 3))
        act = jnp.sum(p[:, :, :, 1, :], axis=(1, 3))
        return box, act

    box0, act0 = sc_sums(p0_ref[...])
    box1, act1 = sc_sums(p1_ref[...])
    t = t_ref[...]
    box = jnp.concatenate([box0, box1], axis=0) + t[:, :, 0]
    act = jnp.concatenate([act0, act1], axis=0) + t[:, :, 1]
    over = act - _MAXIMUM * box
    under = _MINIMUM * box - act
    err = (jnp.where(over >= 0, over * over, 0.0)
           + jnp.where(under >= 0, under * under, 0.0))
    out_ref[0, 0] = jnp.sum(err)


def kernel(logits, box_masks):
    B, C, W, H = logits.shape
    N = box_masks.shape[2]
    Cf = C - 1
    Wtc = W - _SC_ROWS

    tc_part = pl.pallas_call(
        _tc_main_body,
        grid=(B, Cf),
        in_specs=[
            pl.BlockSpec((1, 1, Wtc, H),
                         lambda b, c: (b, c + 1, W // (W - _SC_ROWS) - 1, 0)),
            pl.BlockSpec((1, 1, N, Wtc, H),
                         lambda b, c: (b, c + 1, 0, W // (W - _SC_ROWS) - 1,
                                       0)),
        ],
        out_specs=pl.BlockSpec((1, 1, N, 2), lambda b, c: (b, c, 0, 0)),
        out_shape=jax.ShapeDtypeStruct((B, Cf, N, 2), jnp.float32),
    )(logits, box_masks)

    p0, p1 = _sc_stage(logits, box_masks, B, C, N, W, H)

    n_half = B * Cf // 2
    p0 = p0.reshape(n_half, 4, N, 2, _LANES)
    p1 = p1.reshape(n_half, 4, N, 2, _LANES)
    tc_part = tc_part.reshape(B * Cf, N, 2)

    out = pl.pallas_call(
        _combine_body,
        out_specs=pl.BlockSpec(memory_space=pltpu.SMEM),
        out_shape=jax.ShapeDtypeStruct((1, 1), jnp.float32),
    )(p0, p1, tc_part)
    return out[0, 0] / float(Cf * W * H)


# speedup vs baseline: 2.0877x; 2.0877x over previous
import jax
import jax.numpy as jnp
from jax.experimental import pallas as pl
from jax.experimental.pallas import tpu as pltpu

_MINIMUM = 0.3
_MAXIMUM = 0.9


def _body(l_ref, m_ref, out_ref):
    b = pl.program_id(0)
    c = pl.program_id(1)
    h = pl.program_id(2)

    @pl.when((b == 0) & (c == 0) & (h == 0))
    def _init():
        out_ref[0, 0] = 0.0

    l = l_ref[0, 0]
    m = m_ref[0, 0]
    box = jnp.sum(m, axis=(1, 2))
    act = jnp.sum(m * l[None, :, :], axis=(1, 2))
    over = act - _MAXIMUM * box
    under = _MINIMUM * box - act
    err = (jnp.where(over >= 0, over * over, 0.0)
           + jnp.where(under >= 0, under * under, 0.0))
    out_ref[0, 0] += jnp.sum(err)


def kernel(logits, box_masks):
    B, C, W, H = logits.shape
    N = box_masks.shape[2]
    Cf = C - 1

    Nh = N // 2
    out = pl.pallas_call(
        _body,
        grid=(B, Cf, 2),
        in_specs=[
            pl.BlockSpec((1, 1, W, H), lambda b, c, h: (b, c + 1, 0, 0)),
            pl.BlockSpec((1, 1, Nh, W, H),
                         lambda b, c, h: (b, c + 1, h, 0, 0)),
        ],
        out_specs=pl.BlockSpec(memory_space=pltpu.SMEM),
        out_shape=jax.ShapeDtypeStruct((1, 1), jnp.float32),
    )(logits, box_masks)
    return out[0, 0] / float(Cf * W * H)


# speedup vs baseline: 2.7186x; 1.3021x over previous
import jax
import jax.numpy as jnp
from jax.experimental import pallas as pl
from jax.experimental.pallas import tpu as pltpu

_MINIMUM = 0.3
_MAXIMUM = 0.9


def _body(l_ref, m_ref, out_ref):
    b = pl.program_id(0)
    c = pl.program_id(1)

    @pl.when((b == 0) & (c == 0))
    def _init():
        out_ref[0, 0] = 0.0

    l = l_ref[0, 0]
    m = m_ref[0, 0]
    box = jnp.sum(m, axis=(1, 2))
    act = jnp.sum(m * l[None, :, :], axis=(1, 2))
    over = act - _MAXIMUM * box
    under = _MINIMUM * box - act
    err = (jnp.where(over >= 0, over * over, 0.0)
           + jnp.where(under >= 0, under * under, 0.0))
    out_ref[0, 0] += jnp.sum(err)


def kernel(logits, box_masks):
    B, C, W, H = logits.shape
    N = box_masks.shape[2]
    Cf = C - 1

    out = pl.pallas_call(
        _body,
        grid=(B, Cf),
        in_specs=[
            pl.BlockSpec((1, 1, W, H), lambda b, c: (b, c + 1, 0, 0)),
            pl.BlockSpec((1, 1, N, W, H), lambda b, c: (b, c + 1, 0, 0, 0)),
        ],
        out_specs=pl.BlockSpec(memory_space=pltpu.SMEM),
        out_shape=jax.ShapeDtypeStruct((1, 1), jnp.float32),
    )(logits, box_masks)
    return out[0, 0] / float(Cf * W * H)
